# baseline (speedup 1.0000x reference)
import functools

import jax
import jax.numpy as jnp
from jax.experimental import pallas as pl
from jax.experimental.pallas import tpu as pltpu

_B = 4
_N = 768
_M = 768
_EMB = 256
_K2 = 4
_NN_MARGIN = 0.7


def _dot(a, b, ca, cb):
    return jax.lax.dot_general(
        a, b, (((ca,), (cb,)), ((), ())), preferred_element_type=jnp.float32
    )


def _norm_axis(x, axis, scale=1.0):
    n = x.shape[axis]
    mean = jnp.mean(x, axis=axis, keepdims=True)
    sumsq = jnp.sum(x * x, axis=axis, keepdims=True)
    var = jnp.maximum(sumsq - n * mean * mean, 0.0) / (n - 1)
    return (x - mean) * (scale / (jnp.sqrt(var) + 1e-06))


def _softmax_axis(x, axis):
    m = jnp.max(x, axis=axis, keepdims=True)
    e = jnp.exp(x - m)
    return e / jnp.sum(e, axis=axis, keepdims=True)


def _knn_indicator(points):
    inner = _dot(points, points, 0, 0)
    aa = jnp.sum(points * points, axis=0)
    d = -2.0 * inner + aa[:, None] + aa[None, :]
    big = jnp.float32(3.4e38)
    mn = jnp.min(d, axis=1, keepdims=True)
    mn_first = mn
    for _ in range(_K2 - 1):
        mn = jnp.min(jnp.where(d <= mn, big, d), axis=1, keepdims=True)
    return ((d <= mn) & (d > mn_first)).astype(jnp.float32)


def _rrp_kernel(ref_ref, src_ref, embw_ref, embb_ref, w1_ref, b1_ref,
                w2_ref, b2_ref, out_ref):
    ref_pts = ref_ref[0]
    src_pts = src_ref[0]
    emb_w = embw_ref[...]
    emb_b = embb_ref[...]
    w1 = w1_ref[...]
    b1 = b1_ref[...]
    w2 = w2_ref[...]
    b2 = b2_ref[...]

    src_emb = jnp.tanh(_dot(emb_w, src_pts, 1, 0) + emb_b[0][:, None])
    ref_emb = jnp.tanh(_dot(emb_w, ref_pts, 1, 0) + emb_b[0][:, None])

    aa = jnp.sum(src_emb * src_emb, axis=0)
    bb = jnp.sum(ref_emb * ref_emb, axis=0)
    inner2 = _dot(src_emb * jnp.float32(-2.0), ref_emb, 0, 0)
    dmap = inner2 + (aa[:, None] + bb[None, :])

    x = _dot(w1, dmap, 1, 1) + b1[0][:, None]
    x = jnp.maximum(x, 0.0)
    x = _norm_axis(x, 0)
    x = _dot(w2, x, 1, 0) + b2[0][:, None]
    x = _norm_axis(x, 1, scale=8.0)
    x = _softmax_axis(x, 1)
    scores_t = x

    s_src = _knn_indicator(src_pts)
    s_ref = _knn_indicator(ref_pts)

    sk_sum = _dot(s_src.astype(jnp.bfloat16), scores_t.astype(jnp.bfloat16), 1, 1)
    agg = _dot(sk_sum.astype(jnp.bfloat16), s_ref.astype(jnp.bfloat16), 1, 1)
    src_knn = agg * jnp.float32(1.0 / (_K2 - 1))

    refined = jnp.exp(_NN_MARGIN - src_knn) * dmap
    e = jnp.exp(-refined)
    rs = jnp.sum(e, axis=1)
    out_ref[0] = _dot(ref_pts, e, 1, 1) / rs[None, :]


@jax.jit
def kernel(ref_points, src_points, emb_W, emb_b, ns_W1, ns_b1, ns_W2, ns_b2):
    b = ref_points.shape[0]
    emb_b2d = emb_b.reshape(1, -1)
    b1 = ns_b1.reshape(1, -1)
    b2 = ns_b2.reshape(1, -1)
    grid = (b,)
    out = pl.pallas_call(
        _rrp_kernel,
        grid=grid,
        in_specs=[
            pl.BlockSpec((1, 3, _M), lambda i: (i, 0, 0)),
            pl.BlockSpec((1, 3, _N), lambda i: (i, 0, 0)),
            pl.BlockSpec((_EMB, 3), lambda i: (0, 0)),
            pl.BlockSpec((1, _EMB), lambda i: (0, 0)),
            pl.BlockSpec((_M, _M), lambda i: (0, 0)),
            pl.BlockSpec((1, _M), lambda i: (0, 0)),
            pl.BlockSpec((_M, _M), lambda i: (0, 0)),
            pl.BlockSpec((1, _M), lambda i: (0, 0)),
        ],
        out_specs=pl.BlockSpec((1, 3, _N), lambda i: (i, 0, 0)),
        out_shape=jax.ShapeDtypeStruct((b, 3, _N), jnp.float32),
        compiler_params=pltpu.CompilerParams(
            vmem_limit_bytes=100 * 1024 * 1024,
        ),
    )(ref_points, src_points, emb_W, emb_b2d, ns_W1, b1, ns_W2, b2)
    return out


# speedup vs baseline: 1.0073x; 1.0073x over previous
import functools

import jax
import jax.numpy as jnp
from jax.experimental import pallas as pl
from jax.experimental.pallas import tpu as pltpu

_B = 4
_N = 768
_M = 768
_EMB = 256
_K2 = 4
_NN_MARGIN = 0.7


def _dot(a, b, ca, cb):
    return jax.lax.dot_general(
        a, b, (((ca,), (cb,)), ((), ())), preferred_element_type=jnp.float32
    )


def _norm_axis(x, axis, scale=1.0):
    n = x.shape[axis]
    mean = jnp.mean(x, axis=axis, keepdims=True)
    sumsq = jnp.sum(x * x, axis=axis, keepdims=True)
    var = jnp.maximum(sumsq - n * mean * mean, 0.0) / (n - 1)
    return (x - mean) * (scale / (jnp.sqrt(var) + 1e-06))


def _softmax_axis(x, axis):
    m = jnp.max(x, axis=axis, keepdims=True)
    e = jnp.exp(x - m)
    return e / jnp.sum(e, axis=axis, keepdims=True)


def _knn_indicator(points):
    inner = _dot(points, points, 0, 0)
    aa = jnp.sum(points * points, axis=0)
    d = -2.0 * inner + aa[:, None] + aa[None, :]
    big = jnp.float32(3.4e38)
    mn = jnp.min(d, axis=1, keepdims=True)
    mn_first = mn
    for _ in range(_K2 - 1):
        mn = jnp.min(jnp.where(d <= mn, big, d), axis=1, keepdims=True)
    return ((d <= mn) & (d > mn_first)).astype(jnp.float32)


def _rrp_kernel(ref_ref, src_ref, embw_ref, embb_ref, w1_ref, b1_ref,
                w2_ref, b2_ref, out_ref):
    ref_pts = ref_ref[0]
    src_pts = src_ref[0]
    emb_w = embw_ref[...]
    emb_b = embb_ref[...]
    w1 = w1_ref[...]
    b1 = b1_ref[...]
    w2 = w2_ref[...]
    b2 = b2_ref[...]

    src_emb = jnp.tanh(_dot(emb_w, src_pts, 1, 0) + emb_b[0][:, None])
    ref_emb = jnp.tanh(_dot(emb_w, ref_pts, 1, 0) + emb_b[0][:, None])

    aa = jnp.sum(src_emb * src_emb, axis=0)
    bb = jnp.sum(ref_emb * ref_emb, axis=0)
    inner2 = _dot(src_emb * jnp.float32(-2.0), ref_emb, 0, 0)
    dmap = inner2 + (aa[:, None] + bb[None, :])

    x = _dot(w1, dmap, 1, 1) + b1[0][:, None]
    x = jnp.maximum(x, 0.0)
    x = _norm_axis(x, 0)
    x = _dot(w2, x, 1, 0) + b2[0][:, None]
    x = _norm_axis(x, 1, scale=8.0)
    x = _softmax_axis(x, 1)
    scores_t = x

    s_src = _knn_indicator(src_pts)
    s_ref = _knn_indicator(ref_pts)

    sk_sum = _dot(s_src, scores_t, 1, 1)
    agg = _dot(sk_sum, s_ref, 1, 1)
    src_knn = agg * jnp.float32(1.0 / (_K2 - 1))

    refined = jnp.exp(_NN_MARGIN - src_knn) * dmap
    e = jnp.exp(-refined)
    rs = jnp.sum(e, axis=1)
    out_ref[0] = _dot(ref_pts, e, 1, 1) / rs[None, :]


@jax.jit
def kernel(ref_points, src_points, emb_W, emb_b, ns_W1, ns_b1, ns_W2, ns_b2):
    b = ref_points.shape[0]
    emb_b2d = emb_b.reshape(1, -1)
    b1 = ns_b1.reshape(1, -1)
    b2 = ns_b2.reshape(1, -1)
    grid = (b,)
    out = pl.pallas_call(
        _rrp_kernel,
        grid=grid,
        in_specs=[
            pl.BlockSpec((1, 3, _M), lambda i: (i, 0, 0)),
            pl.BlockSpec((1, 3, _N), lambda i: (i, 0, 0)),
            pl.BlockSpec((_EMB, 3), lambda i: (0, 0)),
            pl.BlockSpec((1, _EMB), lambda i: (0, 0)),
            pl.BlockSpec((_M, _M), lambda i: (0, 0)),
            pl.BlockSpec((1, _M), lambda i: (0, 0)),
            pl.BlockSpec((_M, _M), lambda i: (0, 0)),
            pl.BlockSpec((1, _M), lambda i: (0, 0)),
        ],
        out_specs=pl.BlockSpec((1, 3, _N), lambda i: (i, 0, 0)),
        out_shape=jax.ShapeDtypeStruct((b, 3, _N), jnp.float32),
        compiler_params=pltpu.CompilerParams(
            vmem_limit_bytes=100 * 1024 * 1024,
        ),
    )(ref_points, src_points, emb_W, emb_b2d, ns_W1, b1, ns_W2, b2)
    return out
